# baseline (speedup 1.0000x reference)
import numpy as np
import jax
import jax.numpy as jnp
from jax import lax
from jax.experimental import pallas as pl
from jax.experimental.pallas import tpu as pltpu

_N = 9
_NN = 81
_NP = 128
_DEG = 20
_NEP = _DEG * _NP
_HID = 96
_NCLS = 10
_STEPS = 4
_TB = 32


def _edge_table():
    edges = []
    for i in range(_NN):
        r, c = divmod(i, _N)
        row = []
        for j in range(_NN):
            if j == i:
                continue
            r2, c2 = divmod(j, _N)
            if r2 == r or c2 == c or (r2 // 3 == r // 3 and c2 // 3 == c // 3):
                row.append(j)
        edges.append(row)
    return np.asarray(edges, dtype=np.int32)


_E = _edge_table()

_GT = np.zeros((_NP, _NEP), np.float32)
for _i in range(_NN):
    for _k in range(_DEG):
        _GT[_E[_i, _k], _k * _NP + _i] = 1.0

_IDX = np.zeros((_NEP,), np.int32)
for _i in range(_NN):
    for _k in range(_DEG):
        _IDX[_k * _NP + _i] = _E[_i, _k]


def _rrn_body(g_ref, c0_ref, mask_ref, embT_ref,
              ib0, iW1, ib1, iW2, ib2,
              rWs, rb0, rW1, rb1, rW2, rb2,
              gWa, gWb, gb0, gW1, gb1, gW2, gb2,
              Wih, Whh, bsum,
              oW0, ob0, oW1, ob1, oW2, ob2,
              GT_ref, out_ref):
    f32 = jnp.float32
    H = _HID

    def dot(a, b):
        return jnp.dot(a, b, preferred_element_type=f32)

    def dot16(a, b):
        return jnp.dot(a.astype(jnp.bfloat16), b.astype(jnp.bfloat16),
                       preferred_element_type=f32)

    g = jnp.concatenate([g_ref[t] for t in range(_TB)], axis=1)
    oh = (lax.broadcasted_iota(jnp.int32, (_NCLS, _TB * _NP), 0) == g).astype(f32)
    x = jnp.maximum(dot16(embT_ref[...], oh) + ib0[...], 0.0)
    x = jnp.maximum(dot16(iW1[...], x) + ib1[...], 0.0)
    X = dot16(iW2[...], x) + ib2[...]

    h = X
    c = jnp.concatenate([c0_ref[t] for t in range(_TB)], axis=1)
    gaX = dot16(gWa[...], X) + gb0[...]

    for step in range(_STEPS):
        mv = mask_ref[0, step]
        ab = dot16(rWs[...], h)
        aggs = []
        for t in range(_TB):
            a16 = (ab[0:H, t * _NP:(t + 1) * _NP]
                   + rb0[...]).astype(jnp.bfloat16)
            b_t = ab[H:2 * H, t * _NP:(t + 1) * _NP]
            pre = (pltpu.repeat(a16, _DEG, axis=1)
                   + dot16(b_t, GT_ref[...]).astype(jnp.bfloat16))
            m = jnp.maximum(pre, jnp.bfloat16(0.0))
            m = jnp.maximum(dot16(rW1[...], m) + rb1[...], 0.0)
            agg = m[:, 0:_NP]
            for k in range(1, _DEG):
                agg = agg + m[:, k * _NP:(k + 1) * _NP]
            aggs.append(agg)
        Magg = jnp.concatenate(aggs, axis=1)
        M = dot16(rW2[...], Magg) + rb2[...]

        gx = jnp.maximum(gaX + dot16(gWb[...], M), 0.0)
        gx = jnp.maximum(dot16(gW1[...], gx) + gb1[...], 0.0)
        gin = dot16(gW2[...], gx) + gb2[...]

        gates = dot16(Wih[...], gin) + dot16(Whh[...], h) + bsum[...]
        ig = jax.nn.sigmoid(gates[0:H])
        fg = jax.nn.sigmoid(gates[H:2 * H])
        gg = jnp.tanh(gates[2 * H:3 * H])
        og = jax.nn.sigmoid(gates[3 * H:4 * H])
        c_new = fg * c + ig * gg
        h_new = og * jnp.tanh(c_new)
        c = mv * c_new + (1.0 - mv) * c
        h = mv * h_new + (1.0 - mv) * h

        r = jnp.maximum(dot16(oW0[...], h) + ob0[...], 0.0)
        r = jnp.maximum(dot16(oW1[...], r) + ob1[...], 0.0)
        r = dot16(oW2[...], r) + ob2[...]
        for t in range(_TB):
            out_ref[step, t] = r[:, t * _NP:(t + 1) * _NP]


def kernel(grids, iters, c0, emb, in_params, rel_params, g_params, lstm_params, r_params):
    B = grids.shape[0]
    f32 = jnp.float32

    grids_p = jnp.pad(grids.astype(jnp.int32), ((0, 0), (0, _NP - _NN)),
                      constant_values=_NCLS).reshape(B, 1, _NP)
    c0t = jnp.pad(c0.reshape(B, _NN, _HID).transpose(0, 2, 1),
                  ((0, 0), (0, 0), (0, _NP - _NN)))
    mask = (jnp.arange(_STEPS) < iters).astype(f32).reshape(1, _STEPS)
    embT = jnp.dot(in_params[0][0], emb.T).astype(f32)

    def col(b):
        return b.reshape(-1, 1).astype(f32)

    iW0, ib0 = in_params[0]
    iW1, ib1 = in_params[1]
    iW2, ib2 = in_params[2]
    rW0, rb0 = rel_params[0]
    rW1, rb1 = rel_params[1]
    rW2, rb2 = rel_params[2]
    rWs = jnp.concatenate([rW0[:, :_HID], rW0[:, _HID:]], axis=0)
    gW0, gb0 = g_params[0]
    gW1, gb1 = g_params[1]
    gW2, gb2 = g_params[2]
    gWa, gWb = gW0[:, :_HID], gW0[:, _HID:]
    Wih, Whh, bih, bhh = lstm_params
    bsum = col(bih + bhh)
    oW0, ob0 = r_params[0]
    oW1, ob1 = r_params[1]
    oW2, ob2 = r_params[2]

    rep = lambda i: (0, 0)
    full = lambda a: pl.BlockSpec(a.shape, rep)

    GT = jnp.asarray(_GT)

    operands = [
        grids_p, c0t, mask, embT,
        col(ib0), iW1, col(ib1), iW2, col(ib2),
        rWs, col(rb0), rW1, col(rb1), rW2, col(_DEG * rb2),
        gWa, gWb, col(gb0), gW1, col(gb1), gW2, col(gb2),
        Wih, Whh, bsum,
        oW0, col(ob0), oW1, col(ob1), oW2, col(ob2),
        GT,
    ]
    in_specs = [
        pl.BlockSpec((_TB, 1, _NP), lambda i: (i, 0, 0)),
        pl.BlockSpec((_TB, _HID, _NP), lambda i: (i, 0, 0)),
    ] + [full(a) for a in operands[2:]]

    out = pl.pallas_call(
        _rrn_body,
        grid=(B // _TB,),
        in_specs=in_specs,
        out_specs=pl.BlockSpec((_STEPS, _TB, _N, _NP), lambda i: (0, i, 0, 0)),
        out_shape=jax.ShapeDtypeStruct((_STEPS, B, _N, _NP), f32),
        compiler_params=pltpu.CompilerParams(
            dimension_semantics=("parallel",),
        ),
    )(*operands)

    return out[:, :, :, :_NN].transpose(0, 1, 3, 2)


# speedup vs baseline: 1.1318x; 1.1318x over previous
import numpy as np
import jax
import jax.numpy as jnp
from jax import lax
from jax.experimental import pallas as pl
from jax.experimental.pallas import tpu as pltpu

_N = 9
_NN = 81
_NP = 128
_DEG = 20
_NEP = _DEG * _NP
_HID = 96
_NCLS = 10
_STEPS = 4
_TB = 16


def _edge_table():
    edges = []
    for i in range(_NN):
        r, c = divmod(i, _N)
        row = []
        for j in range(_NN):
            if j == i:
                continue
            r2, c2 = divmod(j, _N)
            if r2 == r or c2 == c or (r2 // 3 == r // 3 and c2 // 3 == c // 3):
                row.append(j)
        edges.append(row)
    return np.asarray(edges, dtype=np.int32)


_E = _edge_table()

_GT = np.zeros((_NP, _NEP), np.float32)
for _i in range(_NN):
    for _k in range(_DEG):
        _GT[_E[_i, _k], _k * _NP + _i] = 1.0

_IDX = np.zeros((_NEP,), np.int32)
for _i in range(_NN):
    for _k in range(_DEG):
        _IDX[_k * _NP + _i] = _E[_i, _k]


def _rrn_body(g_ref, c0_ref, mask_ref, embT_ref,
              ib0, iW1, ib1, iW2, ib2,
              rWs, rb0, rW1, rb1, rW2, rb2,
              gWa, gWb, gb0, gW1, gb1, gW2, gb2,
              Wih, Whh, bsum,
              oW0, ob0, oW1, ob1, oW2, ob2,
              GT_ref, out_ref):
    f32 = jnp.float32
    H = _HID

    def dot(a, b):
        return jnp.dot(a, b, preferred_element_type=f32)

    def dot16(a, b):
        return jnp.dot(a.astype(jnp.bfloat16), b.astype(jnp.bfloat16),
                       preferred_element_type=f32)

    g = jnp.concatenate([g_ref[t] for t in range(_TB)], axis=1)
    oh = (lax.broadcasted_iota(jnp.int32, (_NCLS, _TB * _NP), 0) == g).astype(f32)
    x = jnp.maximum(dot16(embT_ref[...], oh) + ib0[...], 0.0)
    x = jnp.maximum(dot16(iW1[...], x) + ib1[...], 0.0)
    X = dot16(iW2[...], x) + ib2[...]

    h = X
    c = jnp.concatenate([c0_ref[t] for t in range(_TB)], axis=1)
    gaX = dot16(gWa[...], X) + gb0[...]

    for step in range(_STEPS):
        mv = mask_ref[0, step]
        ab = dot16(rWs[...], h)
        aggs = []
        for t in range(_TB):
            a16 = (ab[0:H, t * _NP:(t + 1) * _NP]
                   + rb0[...]).astype(jnp.bfloat16)
            b_t = ab[H:2 * H, t * _NP:(t + 1) * _NP]
            pre = (pltpu.repeat(a16, _DEG, axis=1)
                   + dot16(b_t, GT_ref[...]).astype(jnp.bfloat16))
            m = jnp.maximum(pre, jnp.bfloat16(0.0))
            m = jnp.maximum(dot16(rW1[...], m) + rb1[...], 0.0)
            agg = m[:, 0:_NP]
            for k in range(1, _DEG):
                agg = agg + m[:, k * _NP:(k + 1) * _NP]
            aggs.append(agg)
        Magg = jnp.concatenate(aggs, axis=1)
        M = dot16(rW2[...], Magg) + rb2[...]

        gx = jnp.maximum(gaX + dot16(gWb[...], M), 0.0)
        gx = jnp.maximum(dot16(gW1[...], gx) + gb1[...], 0.0)
        gin = dot16(gW2[...], gx) + gb2[...]

        gates = dot16(Wih[...], gin) + dot16(Whh[...], h) + bsum[...]
        ig = jax.nn.sigmoid(gates[0:H])
        fg = jax.nn.sigmoid(gates[H:2 * H])
        gg = jnp.tanh(gates[2 * H:3 * H])
        og = jax.nn.sigmoid(gates[3 * H:4 * H])
        c_new = fg * c + ig * gg
        h_new = og * jnp.tanh(c_new)
        c = mv * c_new + (1.0 - mv) * c
        h = mv * h_new + (1.0 - mv) * h

        r = jnp.maximum(dot16(oW0[...], h) + ob0[...], 0.0)
        r = jnp.maximum(dot16(oW1[...], r) + ob1[...], 0.0)
        r = dot16(oW2[...], r) + ob2[...]
        for t in range(_TB):
            out_ref[step, t] = r[:, t * _NP:(t + 1) * _NP]


def kernel(grids, iters, c0, emb, in_params, rel_params, g_params, lstm_params, r_params):
    B = grids.shape[0]
    f32 = jnp.float32

    grids_p = jnp.pad(grids.astype(jnp.int32), ((0, 0), (0, _NP - _NN)),
                      constant_values=_NCLS).reshape(B, 1, _NP)
    c0t = jnp.pad(c0.reshape(B, _NN, _HID).transpose(0, 2, 1),
                  ((0, 0), (0, 0), (0, _NP - _NN)))
    mask = (jnp.arange(_STEPS) < iters).astype(f32).reshape(1, _STEPS)
    embT = jnp.dot(in_params[0][0], emb.T).astype(f32)

    def col(b):
        return b.reshape(-1, 1).astype(f32)

    iW0, ib0 = in_params[0]
    iW1, ib1 = in_params[1]
    iW2, ib2 = in_params[2]
    rW0, rb0 = rel_params[0]
    rW1, rb1 = rel_params[1]
    rW2, rb2 = rel_params[2]
    rWs = jnp.concatenate([rW0[:, :_HID], rW0[:, _HID:]], axis=0)
    gW0, gb0 = g_params[0]
    gW1, gb1 = g_params[1]
    gW2, gb2 = g_params[2]
    gWa, gWb = gW0[:, :_HID], gW0[:, _HID:]
    Wih, Whh, bih, bhh = lstm_params
    bsum = col(bih + bhh)
    oW0, ob0 = r_params[0]
    oW1, ob1 = r_params[1]
    oW2, ob2 = r_params[2]

    rep = lambda i: (0, 0)
    full = lambda a: pl.BlockSpec(a.shape, rep)

    GT = jnp.asarray(_GT)

    operands = [
        grids_p, c0t, mask, embT,
        col(ib0), iW1, col(ib1), iW2, col(ib2),
        rWs, col(rb0), rW1, col(rb1), rW2, col(_DEG * rb2),
        gWa, gWb, col(gb0), gW1, col(gb1), gW2, col(gb2),
        Wih, Whh, bsum,
        oW0, col(ob0), oW1, col(ob1), oW2, col(ob2),
        GT,
    ]
    in_specs = [
        pl.BlockSpec((_TB, 1, _NP), lambda i: (i, 0, 0)),
        pl.BlockSpec((_TB, _HID, _NP), lambda i: (i, 0, 0)),
    ] + [full(a) for a in operands[2:]]

    out = pl.pallas_call(
        _rrn_body,
        grid=(B // _TB,),
        in_specs=in_specs,
        out_specs=pl.BlockSpec((_STEPS, _TB, _N, _NP), lambda i: (0, i, 0, 0)),
        out_shape=jax.ShapeDtypeStruct((_STEPS, B, _N, _NP), f32),
        compiler_params=pltpu.CompilerParams(
            dimension_semantics=("parallel",),
        ),
    )(*operands)

    return out[:, :, :, :_NN].transpose(0, 1, 3, 2)
